# baseline (speedup 1.0000x reference)
import functools

import jax
import jax.numpy as jnp
from jax import lax
from jax.experimental import pallas as pl
from jax.experimental.pallas import tpu as pltpu
from jax.experimental.pallas import tpu_sc as plsc

B, K, H, W = 32, 256, 512, 512
HW = H * W
NC, NS = 2, 16
KW = 128
KR = K // KW
EPS = 1e-4

@functools.cache
def _make_sc_gather():
    mesh = plsc.VectorSubcoreMesh(
        core_axis_name="c", subcore_axis_name="s",
        num_cores=NC, num_subcores=NS)

    @functools.partial(
        pl.kernel,
        out_type=jax.ShapeDtypeStruct((3, B, KR, KW), jnp.float32),
        mesh=mesh,
        scratch_types=[
            pltpu.VMEM((3 * KR, KW), jnp.int32),
            pltpu.VMEM((3 * KR, KW), jnp.int32),
            pltpu.VMEM((3 * KR, KW), jnp.float32),
            pltpu.SemaphoreType.DMA,
            pltpu.SemaphoreType.DMA,
            pltpu.SemaphoreType.DMA,
        ],
    )
    def _sc_gather(t1, t2, t3, i1, i2, i3, out,
                   idx_v, adj_v, val_v, sem_i, sem_g, sem_o):
        b = lax.axis_index("s") * NC + lax.axis_index("c")
        b_hi = b >> 3
        b_lo = b & 7
        base = b * HW
        tags = (t1, t2, t3)
        inds = (i1, i2, i3)
        ic = [
            pltpu.async_copy(inds[t].at[b_hi, r, b_lo],
                             idx_v.at[t * KR + r], sem_i)
            for t in range(3) for r in range(KR)
        ]
        for c in ic:
            c.wait()
        for row in range(3 * KR):
            for l in range(KW // 16):
                sl = (row, pl.ds(l * 16, 16))
                v = idx_v[sl]
                adj_v[sl] = (
                    (v & 0x3F000)
                    | ((v & 0x180) << 3)
                    | ((v & 0xE00) >> 2)
                    | (v & 0x7F)
                ) + base
        gc = [
            pltpu.async_copy(tags[t].at[adj_v.at[t * KR + r]],
                             val_v.at[t * KR + r], sem_g)
            for t in range(3) for r in range(KR)
        ]
        for c in gc:
            c.wait()
        oc = [
            pltpu.async_copy(val_v.at[pl.ds(t * KR, KR)], out.at[t, b], sem_o)
            for t in range(3)
        ]
        for c in oc:
            c.wait()

    return _sc_gather


BB = 16


_OUTER = (((0,), (0,)), ((), ()))
_ROWSUM = (((1,), (1,)), ((), ()))
NG = B // BB


def _loss_body(tg, mk, pull_ref, push_ref):
    g = pl.program_id(0)
    third = jnp.float32(1.0 / 3.0)
    v = tg[...]
    r0 = jnp.concatenate([v[0, :, 0, :], v[0, :, 1, :]], axis=1)
    r1 = jnp.concatenate([v[1, :, 0, :], v[1, :, 1, :]], axis=1)
    r2 = jnp.concatenate([v[2, :, 0, :], v[2, :, 1, :]], axis=1)
    mean8 = (r0 + r1 + r2) * third
    km8 = mk[...]
    sq8 = (jnp.square(r0 - mean8) + jnp.square(r1 - mean8)
           + jnp.square(r2 - mean8)) * km8
    ones_row = jnp.ones((1, K), jnp.float32)

    num8 = jnp.sum(km8, axis=1, keepdims=True)
    inv8 = 1.0 / (num8 + EPS)
    pull_acc = jnp.sum(jnp.sum(sq8, axis=1, keepdims=True) * inv8)

    sent = (lax.broadcasted_iota(jnp.int32, (1, K), 1).astype(jnp.float32)
            * 64.0 + 1.0e7)
    mrow8 = jnp.where(km8 > 0.0, mean8, sent)
    mcol8 = jnp.where(km8 > 0.0, mean8, sent + 16.0)
    push_scal = jnp.float32(0.0)
    vacc = jnp.zeros((K, 1), jnp.float32)
    for i in range(BB):
        num = num8[i, 0]
        inv2 = 1.0 / ((num - 1.0) * num + EPS)
        mrow = mrow8[i:i + 1, :]
        mcol = lax.dot_general(mcol8[i:i + 1, :], ones_row, _OUTER,
                               preferred_element_type=jnp.float32)
        R = jnp.maximum(1.0 - jnp.abs(mrow - mcol), 0.0)
        rs = lax.dot_general(R, ones_row, _ROWSUM,
                             preferred_element_type=jnp.float32)
        vacc = vacc + rs * inv2
        push_scal += -num * num * inv8[i, 0] * inv2

    @pl.when(g == 0)
    def _init():
        pull_ref[0, 0] = jnp.float32(0.0)
        push_ref[0, 0] = jnp.float32(0.0)

    pull_ref[0, 0] += pull_acc
    push_ref[0, 0] += push_scal + jnp.sum(vacc)


def _make_loss_call():
    return pl.pallas_call(
        _loss_body,
        grid=(NG,),
        in_specs=[
            pl.BlockSpec((3, BB, KR, KW), lambda g: (0, g, 0, 0)),
            pl.BlockSpec((BB, K), lambda g: (g, 0)),
        ],
        out_specs=[
            pl.BlockSpec((1, 1), lambda g: (0, 0),
                         memory_space=pltpu.MemorySpace.SMEM),
            pl.BlockSpec((1, 1), lambda g: (0, 0),
                         memory_space=pltpu.MemorySpace.SMEM),
        ],
        out_shape=[
            jax.ShapeDtypeStruct((1, 1), jnp.float32),
            jax.ShapeDtypeStruct((1, 1), jnp.float32),
        ],
    )


_loss_call = _make_loss_call()


def _tile_view(tag):
    v = tag.reshape(B, H // 8, 8, W // 128, 128)
    v = v.transpose(0, 1, 3, 2, 4)
    return v.reshape(B * HW)


def _ind_view(ind):
    return ind.astype(jnp.int32).reshape(B // 8, 8, KR, KW).transpose(0, 2, 1, 3)


def kernel(tag1, tag2, tag3, ind1, ind2, ind3, mask):
    gathered = _make_sc_gather()(
        _tile_view(tag1),
        _tile_view(tag2),
        _tile_view(tag3),
        _ind_view(ind1),
        _ind_view(ind2),
        _ind_view(ind3),
    )
    maskf = mask.astype(jnp.float32)
    pull, push = _loss_call(gathered, maskf)
    return pull[0, 0], push[0, 0]


# speedup vs baseline: 1.0162x; 1.0162x over previous
import functools

import jax
import jax.numpy as jnp
from jax import lax
from jax.experimental import pallas as pl
from jax.experimental.pallas import tpu as pltpu
from jax.experimental.pallas import tpu_sc as plsc

B, K, H, W = 32, 256, 512, 512
HW = H * W
NC, NS = 2, 16
KW = 128
KR = K // KW
EPS = 1e-4

@functools.cache
def _make_sc_gather():
    mesh = plsc.VectorSubcoreMesh(
        core_axis_name="c", subcore_axis_name="s",
        num_cores=NC, num_subcores=NS)

    @functools.partial(
        pl.kernel,
        out_type=jax.ShapeDtypeStruct((3, B, KR, KW), jnp.float32),
        mesh=mesh,
        scratch_types=[
            pltpu.VMEM((3 * KR, KW), jnp.int32),
            pltpu.VMEM((3 * KR, KW), jnp.int32),
            pltpu.VMEM((3 * KR, KW), jnp.float32),
            pltpu.SemaphoreType.DMA,
            pltpu.SemaphoreType.DMA,
            pltpu.SemaphoreType.DMA,
        ],
    )
    def _sc_gather(t1, t2, t3, i1, i2, i3, out,
                   idx_v, adj_v, val_v, sem_i, sem_g, sem_o):
        b = lax.axis_index("s") * NC + lax.axis_index("c")
        b_hi = b >> 3
        b_lo = b & 7
        base = b * HW
        tags = (t1, t2, t3)
        inds = (i1, i2, i3)
        ic = [
            pltpu.async_copy(inds[t].at[b_hi, r, b_lo],
                             idx_v.at[t * KR + r], sem_i)
            for t in range(3) for r in range(KR)
        ]
        for c in ic:
            c.wait()
        for row in range(3 * KR):
            for l in range(KW // 16):
                sl = (row, pl.ds(l * 16, 16))
                v = idx_v[sl]
                adj_v[sl] = (
                    (v & 0x3F000)
                    | ((v & 0x180) << 3)
                    | ((v & 0xE00) >> 2)
                    | (v & 0x7F)
                ) + base
        gc = [
            pltpu.async_copy(tags[t].at[adj_v.at[t * KR + r]],
                             val_v.at[t * KR + r], sem_g)
            for t in range(3) for r in range(KR)
        ]
        for c in gc:
            c.wait()
        oc = [
            pltpu.async_copy(val_v.at[pl.ds(t * KR, KR)], out.at[t, b], sem_o)
            for t in range(3)
        ]
        for c in oc:
            c.wait()

    return _sc_gather


BB = 16


_OUTER = (((0,), (0,)), ((), ()))
_ROWSUM = (((1,), (1,)), ((), ()))
NG = B // BB


def _loss_body(tg, mk, pull_ref, push_ref):
    g = pl.program_id(0)
    third = jnp.float32(1.0 / 3.0)
    v = tg[...]
    r0 = jnp.concatenate([v[0, :, 0, :], v[0, :, 1, :]], axis=1)
    r1 = jnp.concatenate([v[1, :, 0, :], v[1, :, 1, :]], axis=1)
    r2 = jnp.concatenate([v[2, :, 0, :], v[2, :, 1, :]], axis=1)
    mean8 = (r0 + r1 + r2) * third
    km8 = mk[...]
    sq8 = (jnp.square(r0 - mean8) + jnp.square(r1 - mean8)
           + jnp.square(r2 - mean8)) * km8
    ones_row = jnp.ones((1, K), jnp.float32)

    num8 = jnp.sum(km8, axis=1, keepdims=True)
    inv8 = 1.0 / (num8 + EPS)
    pull_acc = jnp.sum(jnp.sum(sq8, axis=1, keepdims=True) * inv8)

    sent = (lax.broadcasted_iota(jnp.int32, (1, K), 1).astype(jnp.float32)
            * 64.0 + 1.0e7)
    mrow8 = jnp.where(km8 > 0.0, mean8, sent)
    mcol8 = jnp.where(km8 > 0.0, mean8, sent + 16.0)
    push_scal = jnp.float32(0.0)
    acc = jnp.zeros((K, K), jnp.float32)
    for i in range(BB):
        num = num8[i, 0]
        inv2 = 1.0 / ((num - 1.0) * num + EPS)
        mrow = mrow8[i:i + 1, :]
        mcol = lax.dot_general(mcol8[i:i + 1, :], ones_row, _OUTER,
                               preferred_element_type=jnp.float32)
        R = jnp.maximum(1.0 - jnp.abs(mrow - mcol), 0.0)
        acc = acc + R * inv2
        push_scal += -num * num * inv8[i, 0] * inv2

    @pl.when(g == 0)
    def _init():
        pull_ref[0, 0] = jnp.float32(0.0)
        push_ref[0, 0] = jnp.float32(0.0)

    pull_ref[0, 0] += pull_acc
    push_ref[0, 0] += push_scal + jnp.sum(acc)


def _make_loss_call():
    return pl.pallas_call(
        _loss_body,
        grid=(NG,),
        in_specs=[
            pl.BlockSpec((3, BB, KR, KW), lambda g: (0, g, 0, 0)),
            pl.BlockSpec((BB, K), lambda g: (g, 0)),
        ],
        out_specs=[
            pl.BlockSpec((1, 1), lambda g: (0, 0),
                         memory_space=pltpu.MemorySpace.SMEM),
            pl.BlockSpec((1, 1), lambda g: (0, 0),
                         memory_space=pltpu.MemorySpace.SMEM),
        ],
        out_shape=[
            jax.ShapeDtypeStruct((1, 1), jnp.float32),
            jax.ShapeDtypeStruct((1, 1), jnp.float32),
        ],
    )


_loss_call = _make_loss_call()


def _tile_view(tag):
    v = tag.reshape(B, H // 8, 8, W // 128, 128)
    v = v.transpose(0, 1, 3, 2, 4)
    return v.reshape(B * HW)


def _ind_view(ind):
    return ind.astype(jnp.int32).reshape(B // 8, 8, KR, KW).transpose(0, 2, 1, 3)


def kernel(tag1, tag2, tag3, ind1, ind2, ind3, mask):
    gathered = _make_sc_gather()(
        _tile_view(tag1),
        _tile_view(tag2),
        _tile_view(tag3),
        _ind_view(ind1),
        _ind_view(ind2),
        _ind_view(ind3),
    )
    maskf = mask.astype(jnp.float32)
    pull, push = _loss_call(gathered, maskf)
    return pull[0, 0], push[0, 0]


# speedup vs baseline: 1.0304x; 1.0140x over previous
import functools

import jax
import jax.numpy as jnp
from jax import lax
from jax.experimental import pallas as pl
from jax.experimental.pallas import tpu as pltpu
from jax.experimental.pallas import tpu_sc as plsc

B, K, H, W = 32, 256, 512, 512
HW = H * W
NC, NS = 2, 16
KW = 128
KR = K // KW
EPS = 1e-4

@functools.cache
def _make_sc_gather():
    mesh = plsc.VectorSubcoreMesh(
        core_axis_name="c", subcore_axis_name="s",
        num_cores=NC, num_subcores=NS)

    @functools.partial(
        pl.kernel,
        out_type=jax.ShapeDtypeStruct((3, B, KR, KW), jnp.float32),
        mesh=mesh,
        scratch_types=[
            pltpu.VMEM((3 * KR, KW), jnp.int32),
            pltpu.VMEM((3 * KR, KW), jnp.int32),
            pltpu.VMEM((3 * KR, KW), jnp.float32),
            pltpu.SemaphoreType.DMA,
            pltpu.SemaphoreType.DMA,
            pltpu.SemaphoreType.DMA,
        ],
    )
    def _sc_gather(t1, t2, t3, i1, i2, i3, out,
                   idx_v, adj_v, val_v, sem_i, sem_g, sem_o):
        b = lax.axis_index("s") * NC + lax.axis_index("c")
        b_hi = b >> 3
        b_lo = b & 7
        base = b * HW
        tags = (t1, t2, t3)
        inds = (i1, i2, i3)
        ic = [
            pltpu.async_copy(inds[t].at[b_hi, r, b_lo],
                             idx_v.at[t * KR + r], sem_i)
            for t in range(3) for r in range(KR)
        ]
        for c in ic:
            c.wait()
        for row in range(3 * KR):
            for l in range(KW // 16):
                sl = (row, pl.ds(l * 16, 16))
                v = idx_v[sl]
                adj_v[sl] = (
                    (v & 0x3F000)
                    | ((v & 0x180) << 3)
                    | ((v & 0xE00) >> 2)
                    | (v & 0x7F)
                ) + base
        gc = [
            pltpu.async_copy(tags[t].at[adj_v.at[t * KR + r]],
                             val_v.at[t * KR + r], sem_g)
            for t in range(3) for r in range(KR)
        ]
        for c in gc:
            c.wait()
        oc = [
            pltpu.async_copy(val_v.at[pl.ds(t * KR, KR)], out.at[t, b], sem_o)
            for t in range(3)
        ]
        for c in oc:
            c.wait()

    return _sc_gather


BB = 32


_OUTER = (((0,), (0,)), ((), ()))
_ROWSUM = (((1,), (1,)), ((), ()))
NG = B // BB


def _loss_body(tg, mk, pull_ref, push_ref):
    g = pl.program_id(0)
    third = jnp.float32(1.0 / 3.0)
    v = tg[...]
    r0 = jnp.concatenate([v[0, :, 0, :], v[0, :, 1, :]], axis=1)
    r1 = jnp.concatenate([v[1, :, 0, :], v[1, :, 1, :]], axis=1)
    r2 = jnp.concatenate([v[2, :, 0, :], v[2, :, 1, :]], axis=1)
    mean8 = (r0 + r1 + r2) * third
    km8 = mk[...]
    sq8 = (jnp.square(r0 - mean8) + jnp.square(r1 - mean8)
           + jnp.square(r2 - mean8)) * km8
    ones_row = jnp.ones((1, K), jnp.float32)

    num8 = jnp.sum(km8, axis=1, keepdims=True)
    inv8 = 1.0 / (num8 + EPS)
    pull_acc = jnp.sum(jnp.sum(sq8, axis=1, keepdims=True) * inv8)

    sent = (lax.broadcasted_iota(jnp.int32, (1, K), 1).astype(jnp.float32)
            * 64.0 + 1.0e7)
    mrow8 = jnp.where(km8 > 0.0, mean8, sent)
    mcol8 = jnp.where(km8 > 0.0, mean8, sent + 16.0)
    push_scal = jnp.float32(0.0)
    acc = jnp.zeros((K, K), jnp.float32)
    for i in range(BB):
        num = num8[i, 0]
        inv2 = 1.0 / ((num - 1.0) * num + EPS)
        mrow = mrow8[i:i + 1, :]
        mcol = lax.dot_general(mcol8[i:i + 1, :], ones_row, _OUTER,
                               preferred_element_type=jnp.float32)
        R = jnp.maximum(1.0 - jnp.abs(mrow - mcol), 0.0)
        acc = acc + R * inv2
        push_scal += -num * num * inv8[i, 0] * inv2

    @pl.when(g == 0)
    def _init():
        pull_ref[0, 0] = jnp.float32(0.0)
        push_ref[0, 0] = jnp.float32(0.0)

    pull_ref[0, 0] += pull_acc
    push_ref[0, 0] += push_scal + jnp.sum(acc)


def _make_loss_call():
    return pl.pallas_call(
        _loss_body,
        grid=(NG,),
        in_specs=[
            pl.BlockSpec((3, BB, KR, KW), lambda g: (0, g, 0, 0)),
            pl.BlockSpec((BB, K), lambda g: (g, 0)),
        ],
        out_specs=[
            pl.BlockSpec((1, 1), lambda g: (0, 0),
                         memory_space=pltpu.MemorySpace.SMEM),
            pl.BlockSpec((1, 1), lambda g: (0, 0),
                         memory_space=pltpu.MemorySpace.SMEM),
        ],
        out_shape=[
            jax.ShapeDtypeStruct((1, 1), jnp.float32),
            jax.ShapeDtypeStruct((1, 1), jnp.float32),
        ],
    )


_loss_call = _make_loss_call()


def _tile_view(tag):
    v = tag.reshape(B, H // 8, 8, W // 128, 128)
    v = v.transpose(0, 1, 3, 2, 4)
    return v.reshape(B * HW)


def _ind_view(ind):
    return ind.astype(jnp.int32).reshape(B // 8, 8, KR, KW).transpose(0, 2, 1, 3)


def kernel(tag1, tag2, tag3, ind1, ind2, ind3, mask):
    gathered = _make_sc_gather()(
        _tile_view(tag1),
        _tile_view(tag2),
        _tile_view(tag3),
        _ind_view(ind1),
        _ind_view(ind2),
        _ind_view(ind3),
    )
    maskf = mask.astype(jnp.float32)
    pull, push = _loss_call(gathered, maskf)
    return pull[0, 0], push[0, 0]


# speedup vs baseline: 1.0416x; 1.0109x over previous
import functools

import jax
import jax.numpy as jnp
from jax import lax
from jax.experimental import pallas as pl
from jax.experimental.pallas import tpu as pltpu
from jax.experimental.pallas import tpu_sc as plsc

B, K, H, W = 32, 256, 512, 512
HW = H * W
NC, NS = 2, 16
KW = 128
KR = K // KW
EPS = 1e-4

@functools.cache
def _make_sc_gather():
    mesh = plsc.VectorSubcoreMesh(
        core_axis_name="c", subcore_axis_name="s",
        num_cores=NC, num_subcores=NS)

    @functools.partial(
        pl.kernel,
        out_type=jax.ShapeDtypeStruct((3, B, KR, KW), jnp.float32),
        mesh=mesh,
        scratch_types=[
            pltpu.VMEM((3 * KR, KW), jnp.int32),
            pltpu.VMEM((3 * KR, KW), jnp.int32),
            pltpu.VMEM((3 * KR, KW), jnp.float32),
            pltpu.SemaphoreType.DMA,
            pltpu.SemaphoreType.DMA,
            pltpu.SemaphoreType.DMA,
            pltpu.SemaphoreType.DMA,
            pltpu.SemaphoreType.DMA,
            pltpu.SemaphoreType.DMA,
            pltpu.SemaphoreType.DMA,
        ],
    )
    def _sc_gather(t1, t2, t3, i1, i2, i3, out, idx_v, adj_v, val_v,
                   si0, si1, si2, sg0, sg1, sg2, sem_o):
        b = lax.axis_index("s") * NC + lax.axis_index("c")
        b_hi = b >> 3
        b_lo = b & 7
        base = b * HW
        tags = (t1, t2, t3)
        inds = (i1, i2, i3)
        sem_i = (si0, si1, si2)
        sem_g = (sg0, sg1, sg2)
        ic = [
            [pltpu.async_copy(inds[t].at[b_hi, r, b_lo],
                              idx_v.at[t * KR + r], sem_i[t])
             for r in range(KR)]
            for t in range(3)
        ]
        gc = []
        for t in range(3):
            for c in ic[t]:
                c.wait()
            for r in range(KR):
                row = t * KR + r
                for l in range(KW // 16):
                    sl = (row, pl.ds(l * 16, 16))
                    v = idx_v[sl]
                    adj_v[sl] = (
                        (v & 0x3F000)
                        | ((v & 0x180) << 3)
                        | ((v & 0xE00) >> 2)
                        | (v & 0x7F)
                    ) + base
            gc.append([
                pltpu.async_copy(tags[t].at[adj_v.at[t * KR + r]],
                                 val_v.at[t * KR + r], sem_g[t])
                for r in range(KR)
            ])
        oc = []
        for t in range(3):
            for c in gc[t]:
                c.wait()
            oc.append(pltpu.async_copy(val_v.at[pl.ds(t * KR, KR)],
                                       out.at[t, b], sem_o))
        for c in oc:
            c.wait()

    return _sc_gather


BB = 32


_OUTER = (((0,), (0,)), ((), ()))
_ROWSUM = (((1,), (1,)), ((), ()))
NG = B // BB


def _loss_body(tg, mk, pull_ref, push_ref):
    g = pl.program_id(0)
    third = jnp.float32(1.0 / 3.0)
    v = tg[...]
    r0 = jnp.concatenate([v[0, :, 0, :], v[0, :, 1, :]], axis=1)
    r1 = jnp.concatenate([v[1, :, 0, :], v[1, :, 1, :]], axis=1)
    r2 = jnp.concatenate([v[2, :, 0, :], v[2, :, 1, :]], axis=1)
    mean8 = (r0 + r1 + r2) * third
    km8 = mk[...]
    sq8 = (jnp.square(r0 - mean8) + jnp.square(r1 - mean8)
           + jnp.square(r2 - mean8)) * km8
    ones_row = jnp.ones((1, K), jnp.float32)

    num8 = jnp.sum(km8, axis=1, keepdims=True)
    inv8 = 1.0 / (num8 + EPS)
    pull_acc = jnp.sum(jnp.sum(sq8, axis=1, keepdims=True) * inv8)

    sent = (lax.broadcasted_iota(jnp.int32, (1, K), 1).astype(jnp.float32)
            * 64.0 + 1.0e7)
    mrow8 = jnp.where(km8 > 0.0, mean8, sent)
    mcol8 = jnp.where(km8 > 0.0, mean8, sent + 16.0)
    push_scal = jnp.float32(0.0)
    acc = jnp.zeros((K, K), jnp.float32)
    for i in range(BB):
        num = num8[i, 0]
        inv2 = 1.0 / ((num - 1.0) * num + EPS)
        mrow = mrow8[i:i + 1, :]
        mcol = lax.dot_general(mcol8[i:i + 1, :], ones_row, _OUTER,
                               preferred_element_type=jnp.float32)
        R = jnp.maximum(1.0 - jnp.abs(mrow - mcol), 0.0)
        acc = acc + R * inv2
        push_scal += -num * num * inv8[i, 0] * inv2

    @pl.when(g == 0)
    def _init():
        pull_ref[0, 0] = jnp.float32(0.0)
        push_ref[0, 0] = jnp.float32(0.0)

    pull_ref[0, 0] += pull_acc
    push_ref[0, 0] += push_scal + jnp.sum(acc)


def _make_loss_call():
    return pl.pallas_call(
        _loss_body,
        grid=(NG,),
        in_specs=[
            pl.BlockSpec((3, BB, KR, KW), lambda g: (0, g, 0, 0)),
            pl.BlockSpec((BB, K), lambda g: (g, 0)),
        ],
        out_specs=[
            pl.BlockSpec((1, 1), lambda g: (0, 0),
                         memory_space=pltpu.MemorySpace.SMEM),
            pl.BlockSpec((1, 1), lambda g: (0, 0),
                         memory_space=pltpu.MemorySpace.SMEM),
        ],
        out_shape=[
            jax.ShapeDtypeStruct((1, 1), jnp.float32),
            jax.ShapeDtypeStruct((1, 1), jnp.float32),
        ],
    )


_loss_call = _make_loss_call()


def _tile_view(tag):
    v = tag.reshape(B, H // 8, 8, W // 128, 128)
    v = v.transpose(0, 1, 3, 2, 4)
    return v.reshape(B * HW)


def _ind_view(ind):
    return ind.astype(jnp.int32).reshape(B // 8, 8, KR, KW).transpose(0, 2, 1, 3)


def kernel(tag1, tag2, tag3, ind1, ind2, ind3, mask):
    gathered = _make_sc_gather()(
        _tile_view(tag1),
        _tile_view(tag2),
        _tile_view(tag3),
        _ind_view(ind1),
        _ind_view(ind2),
        _ind_view(ind3),
    )
    maskf = mask.astype(jnp.float32)
    pull, push = _loss_call(gathered, maskf)
    return pull[0, 0], push[0, 0]


# speedup vs baseline: 1.0449x; 1.0032x over previous
import functools

import jax
import jax.numpy as jnp
from jax import lax
from jax.experimental import pallas as pl
from jax.experimental.pallas import tpu as pltpu
from jax.experimental.pallas import tpu_sc as plsc

B, K, H, W = 32, 256, 512, 512
HW = H * W
NC, NS = 2, 16
KW = 128
KR = K // KW
EPS = 1e-4

@functools.cache
def _make_sc_gather():
    mesh = plsc.VectorSubcoreMesh(
        core_axis_name="c", subcore_axis_name="s",
        num_cores=NC, num_subcores=NS)

    @functools.partial(
        pl.kernel,
        out_type=jax.ShapeDtypeStruct((3, B, KR, KW), jnp.float32),
        mesh=mesh,
        scratch_types=[
            pltpu.VMEM((3 * KR, KW), jnp.int32),
            pltpu.VMEM((3 * KR, KW), jnp.int32),
            pltpu.VMEM((3 * KR, KW), jnp.float32),
            pltpu.SemaphoreType.DMA,
            pltpu.SemaphoreType.DMA,
            pltpu.SemaphoreType.DMA,
            pltpu.SemaphoreType.DMA,
            pltpu.SemaphoreType.DMA,
            pltpu.SemaphoreType.DMA,
            pltpu.SemaphoreType.DMA,
        ],
    )
    def _sc_gather(t1, t2, t3, i1, i2, i3, out, idx_v, adj_v, val_v,
                   si0, si1, si2, sg0, sg1, sg2, sem_o):
        b = lax.axis_index("s") * NC + lax.axis_index("c")
        b_hi = b >> 3
        b_lo = b & 7
        base = b * HW
        tags = (t1, t2, t3)
        inds = (i1, i2, i3)
        sem_i = (si0, si1, si2)
        sem_g = (sg0, sg1, sg2)
        ic = [
            pltpu.async_copy(inds[t].at[b_hi, :, b_lo],
                             idx_v.at[pl.ds(t * KR, KR)], sem_i[t])
            for t in range(3)
        ]
        gc = []
        for t in range(3):
            ic[t].wait()
            for r in range(KR):
                row = t * KR + r
                for l in range(KW // 16):
                    sl = (row, pl.ds(l * 16, 16))
                    v = idx_v[sl]
                    adj_v[sl] = (
                        (v & 0x3F000)
                        | ((v & 0x180) << 3)
                        | ((v & 0xE00) >> 2)
                        | (v & 0x7F)
                    ) + base
            gc.append([
                pltpu.async_copy(tags[t].at[adj_v.at[t * KR + r]],
                                 val_v.at[t * KR + r], sem_g[t])
                for r in range(KR)
            ])
        oc = []
        for t in range(3):
            for c in gc[t]:
                c.wait()
            oc.append(pltpu.async_copy(val_v.at[pl.ds(t * KR, KR)],
                                       out.at[t, b], sem_o))
        for c in oc:
            c.wait()

    return _sc_gather


BB = 32


_OUTER = (((0,), (0,)), ((), ()))
_ROWSUM = (((1,), (1,)), ((), ()))
NG = B // BB


def _loss_body(tg, mk, pull_ref, push_ref):
    g = pl.program_id(0)
    third = jnp.float32(1.0 / 3.0)
    v = tg[...]
    r0 = jnp.concatenate([v[0, :, 0, :], v[0, :, 1, :]], axis=1)
    r1 = jnp.concatenate([v[1, :, 0, :], v[1, :, 1, :]], axis=1)
    r2 = jnp.concatenate([v[2, :, 0, :], v[2, :, 1, :]], axis=1)
    mean8 = (r0 + r1 + r2) * third
    km8 = mk[...]
    sq8 = (jnp.square(r0 - mean8) + jnp.square(r1 - mean8)
           + jnp.square(r2 - mean8)) * km8
    ones_row = jnp.ones((1, K), jnp.float32)

    num8 = jnp.sum(km8, axis=1, keepdims=True)
    inv8 = 1.0 / (num8 + EPS)
    pull_acc = jnp.sum(jnp.sum(sq8, axis=1, keepdims=True) * inv8)

    sent = (lax.broadcasted_iota(jnp.int32, (1, K), 1).astype(jnp.float32)
            * 64.0 + 1.0e7)
    mrow8 = jnp.where(km8 > 0.0, mean8, sent)
    mcol8 = jnp.where(km8 > 0.0, mean8, sent + 16.0)
    push_scal = jnp.float32(0.0)
    acc = jnp.zeros((K, K), jnp.float32)
    for i in range(BB):
        num = num8[i, 0]
        inv2 = 1.0 / ((num - 1.0) * num + EPS)
        mrow = mrow8[i:i + 1, :]
        mcol = lax.dot_general(mcol8[i:i + 1, :], ones_row, _OUTER,
                               preferred_element_type=jnp.float32)
        R = jnp.maximum(1.0 - jnp.abs(mrow - mcol), 0.0)
        acc = acc + R * inv2
        push_scal += -num * num * inv8[i, 0] * inv2

    @pl.when(g == 0)
    def _init():
        pull_ref[0, 0] = jnp.float32(0.0)
        push_ref[0, 0] = jnp.float32(0.0)

    pull_ref[0, 0] += pull_acc
    push_ref[0, 0] += push_scal + jnp.sum(acc)


def _make_loss_call():
    return pl.pallas_call(
        _loss_body,
        grid=(NG,),
        in_specs=[
            pl.BlockSpec((3, BB, KR, KW), lambda g: (0, g, 0, 0)),
            pl.BlockSpec((BB, K), lambda g: (g, 0)),
        ],
        out_specs=[
            pl.BlockSpec((1, 1), lambda g: (0, 0),
                         memory_space=pltpu.MemorySpace.SMEM),
            pl.BlockSpec((1, 1), lambda g: (0, 0),
                         memory_space=pltpu.MemorySpace.SMEM),
        ],
        out_shape=[
            jax.ShapeDtypeStruct((1, 1), jnp.float32),
            jax.ShapeDtypeStruct((1, 1), jnp.float32),
        ],
    )


_loss_call = _make_loss_call()


def _tile_view(tag):
    v = tag.reshape(B, H // 8, 8, W // 128, 128)
    v = v.transpose(0, 1, 3, 2, 4)
    return v.reshape(B * HW)


def _ind_view(ind):
    return ind.astype(jnp.int32).reshape(B // 8, 8, KR, KW).transpose(0, 2, 1, 3)


def kernel(tag1, tag2, tag3, ind1, ind2, ind3, mask):
    gathered = _make_sc_gather()(
        _tile_view(tag1),
        _tile_view(tag2),
        _tile_view(tag3),
        _ind_view(ind1),
        _ind_view(ind2),
        _ind_view(ind3),
    )
    maskf = mask.astype(jnp.float32)
    pull, push = _loss_call(gathered, maskf)
    return pull[0, 0], push[0, 0]
